# Initial kernel scaffold; baseline (speedup 1.0000x reference)
import jax
import jax.numpy as jnp
from jax.experimental import pallas as pl


def kernel(cloud, W):
    raise NotImplementedError("write your pallas kernel here")


# speedup vs baseline: 4.6255x; 4.6255x over previous
import functools
import jax
import jax.numpy as jnp
from jax import lax
from jax.experimental import pallas as pl

B, C, N, K = 4, 64, 2048, 20
OUT = 128
RT = 256
NW = 32
SEG_PER_W = (B * N) // NW


def _knn_body(x_ref, xt_ref, w_ref, pt_ref, qt_ref, idx_ref):
    b = pl.program_id(0)
    xf = x_ref[0]
    xs = xt_ref[0]
    inner = lax.dot_general(xs, xf, (((0,), (0,)), ((), ())),
                            preferred_element_type=jnp.float32)
    sq = jnp.sum(xf * xf, axis=0, keepdims=True)
    sq_col = jnp.transpose(jnp.sum(xs * xs, axis=0, keepdims=True))
    nd = (2.0 * inner - sq_col) - sq

    iota = lax.broadcasted_iota(jnp.int32, (RT, N), 1)
    neg = jnp.float32(-jnp.inf)
    cols = []
    for _ in range(K):
        m = jnp.max(nd, axis=1, keepdims=True)
        hit = nd == m
        j = jnp.min(jnp.where(hit, iota, N), axis=1, keepdims=True)
        cols.append(j)
        nd = jnp.where(iota == j, neg, nd)
    idx_ref[0] = jnp.concatenate(cols, axis=1) + b * N

    w1 = w_ref[:, :C]
    w21 = w_ref[:, C:] - w1
    pt_ref[0] = lax.dot_general(xs, w1, (((0,), (1,)), ((), ())),
                                preferred_element_type=jnp.float32)
    qt_ref[0] = lax.dot_general(xs, w21, (((0,), (1,)), ((), ())),
                                preferred_element_type=jnp.float32)


_knn_call = pl.pallas_call(
    _knn_body,
    grid=(B, N // RT),
    in_specs=[
        pl.BlockSpec((1, C, N), lambda b, i: (b, 0, 0)),
        pl.BlockSpec((1, C, RT), lambda b, i: (b, 0, i)),
        pl.BlockSpec((OUT, 2 * C), lambda b, i: (0, 0)),
    ],
    out_specs=[
        pl.BlockSpec((1, RT, OUT), lambda b, i: (b, i, 0)),
        pl.BlockSpec((1, RT, OUT), lambda b, i: (b, i, 0)),
        pl.BlockSpec((1, RT, K), lambda b, i: (b, i, 0)),
    ],
    out_shape=[
        jax.ShapeDtypeStruct((B, N, OUT), jnp.float32),
        jax.ShapeDtypeStruct((B, N, OUT), jnp.float32),
        jax.ShapeDtypeStruct((B, N, K), jnp.int32),
    ],
)


def _gather_stats(ptf, qtf, idx):
    idx2 = idx.reshape(B * N, K)
    g = ptf[idx2]
    m = jnp.max(g, axis=1)
    t = jnp.sum(g, axis=1)
    t2 = jnp.sum(g * g, axis=1)
    tw = t.reshape(NW, SEG_PER_W, OUT)
    t2w = t2.reshape(NW, SEG_PER_W, OUT)
    qw = qtf.reshape(NW, SEG_PER_W, OUT)
    parts = jnp.stack(
        [jnp.sum(tw, axis=1), jnp.sum(t2w, axis=1),
         jnp.sum(qw * tw, axis=1)], axis=1)
    return m, parts


def _fin_body(m_ref, q_ref, p_ref, out_ref):
    m = m_ref[0]
    q = q_ref[0]
    p = p_ref[...]
    tsum = jnp.sum(p[:, 0, :], axis=0, keepdims=True)
    psq = jnp.sum(p[:, 1, :], axis=0, keepdims=True)
    cross = jnp.sum(p[:, 2, :], axis=0, keepdims=True)
    qs = jnp.sum(q, axis=0, keepdims=True)
    qs2 = jnp.sum(q * q, axis=0, keepdims=True)
    cnt = jnp.float32(N * K)
    mean = (tsum + K * qs) / cnt
    var = (psq + 2.0 * cross + K * qs2) / cnt - mean * mean
    inv = 1.0 / jnp.sqrt(var + 1e-5)
    z = (m + q - mean) * inv
    out_ref[0] = jnp.where(z >= 0, z, 0.2 * z)


_fin_call = pl.pallas_call(
    _fin_body,
    grid=(B,),
    in_specs=[
        pl.BlockSpec((1, N, OUT), lambda b: (b, 0, 0)),
        pl.BlockSpec((1, N, OUT), lambda b: (b, 0, 0)),
        pl.BlockSpec((NW // B, 3, OUT), lambda b: (b, 0, 0)),
    ],
    out_specs=pl.BlockSpec((1, N, OUT), lambda b: (b, 0, 0)),
    out_shape=jax.ShapeDtypeStruct((B, N, OUT), jnp.float32),
)


def kernel(cloud, W):
    pt, qt, idx = _knn_call(cloud, cloud, W)
    ptf = pt.reshape(B * N, OUT)
    qtf = qt.reshape(B * N, OUT)
    m, parts = _gather_stats(ptf, qtf, idx)
    out = _fin_call(m.reshape(B, N, OUT), qt, parts)
    return jnp.transpose(out, (0, 2, 1))


# speedup vs baseline: 9.5384x; 2.0622x over previous
import functools
import jax
import jax.numpy as jnp
from jax import lax
from jax.experimental import pallas as pl
from jax.experimental.pallas import tpu as pltpu
from jax.experimental.pallas import tpu_sc as plsc

B, C, N, K = 4, 64, 2048, 20
OUT = 128
RT = 256
NC, NS = 2, 16
NW = NC * NS
SEG_PER_W = (B * N) // NW
LANES = 16


def _knn_body(x_ref, xt_ref, w_ref, pt_ref, qt_ref, idx_ref):
    b = pl.program_id(0)
    xf = x_ref[0]
    xs = xt_ref[0]
    inner = lax.dot_general(xs, xf, (((0,), (0,)), ((), ())),
                            preferred_element_type=jnp.float32)
    sq = jnp.sum(xf * xf, axis=0, keepdims=True)
    sq_col = jnp.transpose(jnp.sum(xs * xs, axis=0, keepdims=True))
    nd = (2.0 * inner - sq_col) - sq

    iota = lax.broadcasted_iota(jnp.int32, (RT, N), 1)
    neg = jnp.float32(-jnp.inf)
    cols = []
    for _ in range(K):
        m = jnp.max(nd, axis=1, keepdims=True)
        hit = nd == m
        j = jnp.min(jnp.where(hit, iota, N), axis=1, keepdims=True)
        cols.append(j)
        nd = jnp.where(iota == j, neg, nd)
    idx_ref[0] = jnp.concatenate(cols, axis=1) + b * N

    w1 = w_ref[:, :C]
    w21 = w_ref[:, C:] - w1
    pt_ref[0] = lax.dot_general(xs, w1, (((0,), (1,)), ((), ())),
                                preferred_element_type=jnp.float32)
    qt_ref[0] = lax.dot_general(xs, w21, (((0,), (1,)), ((), ())),
                                preferred_element_type=jnp.float32)


_knn_call = pl.pallas_call(
    _knn_body,
    grid=(B, N // RT),
    in_specs=[
        pl.BlockSpec((1, C, N), lambda b, i: (b, 0, 0)),
        pl.BlockSpec((1, C, RT), lambda b, i: (b, 0, i)),
        pl.BlockSpec((OUT, 2 * C), lambda b, i: (0, 0)),
    ],
    out_specs=[
        pl.BlockSpec((1, RT, OUT), lambda b, i: (b, i, 0)),
        pl.BlockSpec((1, RT, OUT), lambda b, i: (b, i, 0)),
        pl.BlockSpec((1, RT, K), lambda b, i: (b, i, 0)),
    ],
    out_shape=[
        jax.ShapeDtypeStruct((B, N, OUT), jnp.float32),
        jax.ShapeDtypeStruct((B, N, OUT), jnp.float32),
        jax.ShapeDtypeStruct((B, N, K), jnp.int32),
    ],
)


@functools.partial(
    pl.kernel,
    mesh=plsc.VectorSubcoreMesh(core_axis_name="c", subcore_axis_name="s"),
    out_type=[
        jax.ShapeDtypeStruct((B * N, OUT), jnp.float32),
        jax.ShapeDtypeStruct((NW * 3 * OUT,), jnp.float32),
    ],
    scratch_types=[
        pltpu.VMEM((SEG_PER_W * K,), jnp.int32),
        pltpu.VMEM((SEG_PER_W, OUT), jnp.float32),
        pltpu.VMEM((SEG_PER_W, OUT), jnp.float32),
        pltpu.VMEM((2 * K, OUT), jnp.float32),
        pltpu.VMEM((3 * OUT,), jnp.float32),
        pltpu.SemaphoreType.DMA,
    ],
)
def _sc_gather(pt_hbm, idx_hbm, q_hbm, m_hbm, part_hbm,
               idx_v, q_v, m_v, rows_v, acc_v, sem):
    wid = lax.axis_index("s") * NC + lax.axis_index("c")
    base = wid * SEG_PER_W
    pltpu.sync_copy(idx_hbm.at[pl.ds(base * K, SEG_PER_W * K)], idx_v)
    pltpu.sync_copy(q_hbm.at[pl.ds(base, SEG_PER_W)], q_v)
    zero = jnp.zeros((LANES,), jnp.float32)
    for c in range(3 * OUT // LANES):
        acc_v[pl.ds(c * LANES, LANES)] = zero

    def pair_body(p, carry):
        pltpu.async_copy(
            pt_hbm.at[idx_v.at[pl.ds(p * 2 * K, 2 * K)]], rows_v, sem).wait()
        for s in range(2):
            seg = p * 2 + s
            for c in range(OUT // LANES):
                sl = pl.ds(c * LANES, LANES)
                v0 = rows_v[s * K, sl]
                mx = v0
                sm = v0
                sq = v0 * v0
                for j in range(1, K):
                    v = rows_v[s * K + j, sl]
                    mx = jnp.maximum(mx, v)
                    sm = sm + v
                    sq = sq + v * v
                m_v[seg, sl] = mx
                qv = q_v[seg, sl]
                a0 = pl.ds(c * LANES, LANES)
                a1 = pl.ds(OUT + c * LANES, LANES)
                a2 = pl.ds(2 * OUT + c * LANES, LANES)
                acc_v[a0] = acc_v[a0] + sm
                acc_v[a1] = acc_v[a1] + sq
                acc_v[a2] = acc_v[a2] + qv * sm
        return carry

    lax.fori_loop(0, SEG_PER_W // 2, pair_body, 0)
    pltpu.sync_copy(m_v, m_hbm.at[pl.ds(base, SEG_PER_W)])
    pltpu.sync_copy(acc_v, part_hbm.at[pl.ds(wid * 3 * OUT, 3 * OUT)])


def _gather_stats(ptf, qtf, idxf):
    m, parts_flat = _sc_gather(ptf, idxf, qtf)
    return m, parts_flat.reshape(NW, 3, OUT)


def _fin_body(m_ref, q_ref, p_ref, out_ref):
    m = m_ref[0]
    q = q_ref[0]
    p = p_ref[...]
    tsum = jnp.sum(p[:, 0, :], axis=0, keepdims=True)
    psq = jnp.sum(p[:, 1, :], axis=0, keepdims=True)
    cross = jnp.sum(p[:, 2, :], axis=0, keepdims=True)
    qs = jnp.sum(q, axis=0, keepdims=True)
    qs2 = jnp.sum(q * q, axis=0, keepdims=True)
    cnt = jnp.float32(N * K)
    mean = (tsum + K * qs) / cnt
    var = (psq + 2.0 * cross + K * qs2) / cnt - mean * mean
    inv = 1.0 / jnp.sqrt(var + 1e-5)
    z = (m + q - mean) * inv
    out_ref[0] = jnp.where(z >= 0, z, 0.2 * z)


_fin_call = pl.pallas_call(
    _fin_body,
    grid=(B,),
    in_specs=[
        pl.BlockSpec((1, N, OUT), lambda b: (b, 0, 0)),
        pl.BlockSpec((1, N, OUT), lambda b: (b, 0, 0)),
        pl.BlockSpec((NW // B, 3, OUT), lambda b: (b, 0, 0)),
    ],
    out_specs=pl.BlockSpec((1, N, OUT), lambda b: (b, 0, 0)),
    out_shape=jax.ShapeDtypeStruct((B, N, OUT), jnp.float32),
)


def kernel(cloud, W):
    pt, qt, idx = _knn_call(cloud, cloud, W)
    ptf = pt.reshape(B * N, OUT)
    qtf = qt.reshape(B * N, OUT)
    m, parts = _gather_stats(ptf, qtf, idx.reshape(B * N * K))
    out = _fin_call(m.reshape(B, N, OUT), qt, parts)
    return jnp.transpose(out, (0, 2, 1))


# speedup vs baseline: 11.2089x; 1.1751x over previous
import functools
import jax
import jax.numpy as jnp
from jax import lax
from jax.experimental import pallas as pl
from jax.experimental.pallas import tpu as pltpu
from jax.experimental.pallas import tpu_sc as plsc

B, C, N, K = 4, 64, 2048, 20
OUT = 128
RT = 256
NC, NS = 2, 16
NW = NC * NS
SEG_PER_W = (B * N) // NW
LANES = 16
SB = 4


def _knn_body(x_ref, xt_ref, w_ref, pt_ref, qt_ref, idx_ref):
    b = pl.program_id(0)
    xf = x_ref[0]
    xs = xt_ref[0]
    inner = lax.dot_general(xs, xf, (((0,), (0,)), ((), ())),
                            preferred_element_type=jnp.float32)
    sq = jnp.sum(xf * xf, axis=0, keepdims=True)
    sq_col = jnp.transpose(jnp.sum(xs * xs, axis=0, keepdims=True))
    nd = (2.0 * inner - sq_col) - sq

    iota = lax.broadcasted_iota(jnp.int32, (RT, N), 1)
    neg = jnp.float32(-jnp.inf)
    cols = []
    for _ in range(K):
        m = jnp.max(nd, axis=1, keepdims=True)
        hit = nd == m
        j = jnp.min(jnp.where(hit, iota, N), axis=1, keepdims=True)
        cols.append(j)
        nd = jnp.where(iota == j, neg, nd)
    idx_ref[0] = jnp.concatenate(cols, axis=1) + b * N

    w1 = w_ref[:, :C]
    w21 = w_ref[:, C:] - w1
    pt_ref[0] = lax.dot_general(xs, w1, (((0,), (1,)), ((), ())),
                                preferred_element_type=jnp.float32)
    qt_ref[0] = lax.dot_general(xs, w21, (((0,), (1,)), ((), ())),
                                preferred_element_type=jnp.float32)


_knn_call = pl.pallas_call(
    _knn_body,
    grid=(B, N // RT),
    in_specs=[
        pl.BlockSpec((1, C, N), lambda b, i: (b, 0, 0)),
        pl.BlockSpec((1, C, RT), lambda b, i: (b, 0, i)),
        pl.BlockSpec((OUT, 2 * C), lambda b, i: (0, 0)),
    ],
    out_specs=[
        pl.BlockSpec((1, RT, OUT), lambda b, i: (b, i, 0)),
        pl.BlockSpec((1, RT, OUT), lambda b, i: (b, i, 0)),
        pl.BlockSpec((1, RT, K), lambda b, i: (b, i, 0)),
    ],
    out_shape=[
        jax.ShapeDtypeStruct((B, N, OUT), jnp.float32),
        jax.ShapeDtypeStruct((B, N, OUT), jnp.float32),
        jax.ShapeDtypeStruct((B, N, K), jnp.int32),
    ],
)


@functools.partial(
    pl.kernel,
    mesh=plsc.VectorSubcoreMesh(core_axis_name="c", subcore_axis_name="s"),
    out_type=[
        jax.ShapeDtypeStruct((B * N, OUT), jnp.float32),
        jax.ShapeDtypeStruct((NW * 3 * OUT,), jnp.float32),
    ],
    scratch_types=[
        pltpu.VMEM((SEG_PER_W * K,), jnp.int32),
        pltpu.VMEM((SEG_PER_W, OUT), jnp.float32),
        pltpu.VMEM((SEG_PER_W, OUT), jnp.float32),
        pltpu.VMEM((SB * K, OUT), jnp.float32),
        pltpu.VMEM((SB * K, OUT), jnp.float32),
        pltpu.VMEM((3 * OUT,), jnp.float32),
        pltpu.SemaphoreType.DMA,
        pltpu.SemaphoreType.DMA,
    ],
)
def _sc_gather(pt_hbm, idx_hbm, q_hbm, m_hbm, part_hbm,
               idx_v, q_v, m_v, rows0_v, rows1_v, acc_v, sem0, sem1):
    wid = lax.axis_index("s") * NC + lax.axis_index("c")
    base = wid * SEG_PER_W
    nb = SEG_PER_W // SB
    pltpu.sync_copy(idx_hbm.at[pl.ds(base * K, SEG_PER_W * K)], idx_v)
    pltpu.sync_copy(q_hbm.at[pl.ds(base, SEG_PER_W)], q_v)
    zero = jnp.zeros((LANES,), jnp.float32)
    for c in range(3 * OUT // LANES):
        acc_v[pl.ds(c * LANES, LANES)] = zero

    def fire(g, rows_v, sem):
        return pltpu.async_copy(
            pt_hbm.at[idx_v.at[pl.ds(g * SB * K, SB * K)]], rows_v, sem)

    def process(g, rows_v):
        def chunk_body(c, carry):
            sl = pl.ds(c * LANES, LANES)
            for s in range(SB):
                seg = g * SB + s
                v0 = rows_v[s * K, sl]
                mx = v0
                sm = v0
                sq = v0 * v0
                for j in range(1, K):
                    v = rows_v[s * K + j, sl]
                    mx = jnp.maximum(mx, v)
                    sm = sm + v
                    sq = sq + v * v
                m_v[seg, sl] = mx
                qv = q_v[seg, sl]
                a0 = pl.ds(c * LANES, LANES)
                a1 = pl.ds(OUT + c * LANES, LANES)
                a2 = pl.ds(2 * OUT + c * LANES, LANES)
                acc_v[a0] = acc_v[a0] + sm
                acc_v[a1] = acc_v[a1] + sq
                acc_v[a2] = acc_v[a2] + qv * sm
            return carry

        lax.fori_loop(0, OUT // LANES, chunk_body, 0)

    fire(0, rows0_v, sem0)

    def body(h, carry):
        g0 = 2 * h
        g1 = 2 * h + 1

        @pl.when(g1 < nb)
        def _():
            fire(g1, rows1_v, sem1)
        pltpu.make_async_copy(
            pt_hbm.at[idx_v.at[pl.ds(g0 * SB * K, SB * K)]], rows0_v,
            sem0).wait()
        process(g0, rows0_v)

        @pl.when(g1 < nb)
        def _():
            @pl.when(g1 + 1 < nb)
            def _():
                fire(g1 + 1, rows0_v, sem0)
            pltpu.make_async_copy(
                pt_hbm.at[idx_v.at[pl.ds(g1 * SB * K, SB * K)]], rows1_v,
                sem1).wait()
            process(g1, rows1_v)
        return carry

    lax.fori_loop(0, (nb + 1) // 2, body, 0)
    pltpu.sync_copy(m_v, m_hbm.at[pl.ds(base, SEG_PER_W)])
    pltpu.sync_copy(acc_v, part_hbm.at[pl.ds(wid * 3 * OUT, 3 * OUT)])


def _gather_stats(ptf, qtf, idxf):
    m, parts_flat = _sc_gather(ptf, idxf, qtf)
    return m, parts_flat.reshape(NW, 3, OUT)


def _fin_body(m_ref, q_ref, p_ref, out_ref):
    m = m_ref[0]
    q = q_ref[0]
    p = p_ref[...]
    tsum = jnp.sum(p[:, 0, :], axis=0, keepdims=True)
    psq = jnp.sum(p[:, 1, :], axis=0, keepdims=True)
    cross = jnp.sum(p[:, 2, :], axis=0, keepdims=True)
    qs = jnp.sum(q, axis=0, keepdims=True)
    qs2 = jnp.sum(q * q, axis=0, keepdims=True)
    cnt = jnp.float32(N * K)
    mean = (tsum + K * qs) / cnt
    var = (psq + 2.0 * cross + K * qs2) / cnt - mean * mean
    inv = 1.0 / jnp.sqrt(var + 1e-5)
    z = (m + q - mean) * inv
    out_ref[0] = jnp.where(z >= 0, z, 0.2 * z)


_fin_call = pl.pallas_call(
    _fin_body,
    grid=(B,),
    in_specs=[
        pl.BlockSpec((1, N, OUT), lambda b: (b, 0, 0)),
        pl.BlockSpec((1, N, OUT), lambda b: (b, 0, 0)),
        pl.BlockSpec((NW // B, 3, OUT), lambda b: (b, 0, 0)),
    ],
    out_specs=pl.BlockSpec((1, N, OUT), lambda b: (b, 0, 0)),
    out_shape=jax.ShapeDtypeStruct((B, N, OUT), jnp.float32),
)


def kernel(cloud, W):
    pt, qt, idx = _knn_call(cloud, cloud, W)
    ptf = pt.reshape(B * N, OUT)
    qtf = qt.reshape(B * N, OUT)
    m, parts = _gather_stats(ptf, qtf, idx.reshape(B * N * K))
    out = _fin_call(m.reshape(B, N, OUT), qt, parts)
    return jnp.transpose(out, (0, 2, 1))


# speedup vs baseline: 11.4624x; 1.0226x over previous
import functools
import jax
import jax.numpy as jnp
from jax import lax
from jax.experimental import pallas as pl
from jax.experimental.pallas import tpu as pltpu
from jax.experimental.pallas import tpu_sc as plsc

B, C, N, K = 4, 64, 2048, 20
OUT = 128
RT = 256
NC, NS = 2, 16
NW = NC * NS
LANES = 16
SB = 4


def _knn_body(x_ref, xt_ref, w_ref, pt_ref, qt_ref, idx_ref):
    xf = x_ref[0]
    xs = xt_ref[0]
    inner = lax.dot_general(xs, xf, (((0,), (0,)), ((), ())),
                            preferred_element_type=jnp.float32)
    sq = jnp.sum(xf * xf, axis=0, keepdims=True)
    sq_col = jnp.transpose(jnp.sum(xs * xs, axis=0, keepdims=True))
    nd = (2.0 * inner - sq_col) - sq

    iota = lax.broadcasted_iota(jnp.int32, (RT, N), 1)
    neg = jnp.float32(-jnp.inf)
    cols = []
    for _ in range(K):
        m = jnp.max(nd, axis=1, keepdims=True)
        hit = nd == m
        j = jnp.min(jnp.where(hit, iota, N), axis=1, keepdims=True)
        cols.append(j)
        nd = jnp.where(iota == j, neg, nd)
    idx_ref[0] = jnp.concatenate(cols, axis=1)

    w1 = w_ref[:, :C]
    w21 = w_ref[:, C:] - w1
    pt_ref[0] = lax.dot_general(xs, w1, (((0,), (1,)), ((), ())),
                                preferred_element_type=jnp.float32)
    qt_ref[0] = lax.dot_general(xs, w21, (((0,), (1,)), ((), ())),
                                preferred_element_type=jnp.float32)


_knn_call = pl.pallas_call(
    _knn_body,
    grid=(N // RT,),
    in_specs=[
        pl.BlockSpec((1, C, N), lambda i: (0, 0, 0)),
        pl.BlockSpec((1, C, RT), lambda i: (0, 0, i)),
        pl.BlockSpec((OUT, 2 * C), lambda i: (0, 0)),
    ],
    out_specs=[
        pl.BlockSpec((1, RT, OUT), lambda i: (0, i, 0)),
        pl.BlockSpec((1, RT, OUT), lambda i: (0, i, 0)),
        pl.BlockSpec((1, RT, K), lambda i: (0, i, 0)),
    ],
    out_shape=[
        jax.ShapeDtypeStruct((1, N, OUT), jnp.float32),
        jax.ShapeDtypeStruct((1, N, OUT), jnp.float32),
        jax.ShapeDtypeStruct((1, N, K), jnp.int32),
    ],
)

SEG_PER_W = N // NW


@functools.partial(
    pl.kernel,
    mesh=plsc.VectorSubcoreMesh(core_axis_name="c", subcore_axis_name="s"),
    out_type=[
        jax.ShapeDtypeStruct((N, OUT), jnp.float32),
        jax.ShapeDtypeStruct((NW * 3 * OUT,), jnp.float32),
    ],
    scratch_types=[
        pltpu.VMEM((SEG_PER_W * K,), jnp.int32),
        pltpu.VMEM((SEG_PER_W, OUT), jnp.float32),
        pltpu.VMEM((SEG_PER_W, OUT), jnp.float32),
        pltpu.VMEM((SB * K, OUT), jnp.float32),
        pltpu.VMEM((SB * K, OUT), jnp.float32),
        pltpu.VMEM((3 * OUT,), jnp.float32),
        pltpu.SemaphoreType.DMA,
        pltpu.SemaphoreType.DMA,
    ],
)
def _sc_gather(pt_hbm, idx_hbm, q_hbm, m_hbm, part_hbm,
               idx_v, q_v, m_v, rows0_v, rows1_v, acc_v, sem0, sem1):
    wid = lax.axis_index("s") * NC + lax.axis_index("c")
    base = wid * SEG_PER_W
    nb = SEG_PER_W // SB
    pltpu.sync_copy(idx_hbm.at[pl.ds(base * K, SEG_PER_W * K)], idx_v)
    pltpu.sync_copy(q_hbm.at[pl.ds(base, SEG_PER_W)], q_v)
    zero = jnp.zeros((LANES,), jnp.float32)
    for c in range(3 * OUT // LANES):
        acc_v[pl.ds(c * LANES, LANES)] = zero

    def fire(g, rows_v, sem):
        return pltpu.async_copy(
            pt_hbm.at[idx_v.at[pl.ds(g * SB * K, SB * K)]], rows_v, sem)

    def drain(g, rows_v, sem):
        pltpu.make_async_copy(
            pt_hbm.at[idx_v.at[pl.ds(g * SB * K, SB * K)]], rows_v, sem
        ).wait()

    def process(g, rows_v):
        def chunk_body(c, carry):
            sl = pl.ds(c * LANES, LANES)
            for s in range(SB):
                seg = g * SB + s
                v0 = rows_v[s * K, sl]
                mx = v0
                sm = v0
                sq = v0 * v0
                for j in range(1, K):
                    v = rows_v[s * K + j, sl]
                    mx = jnp.maximum(mx, v)
                    sm = sm + v
                    sq = sq + v * v
                m_v[seg, sl] = mx
                qv = q_v[seg, sl]
                a0 = pl.ds(c * LANES, LANES)
                a1 = pl.ds(OUT + c * LANES, LANES)
                a2 = pl.ds(2 * OUT + c * LANES, LANES)
                acc_v[a0] = acc_v[a0] + sm
                acc_v[a1] = acc_v[a1] + sq
                acc_v[a2] = acc_v[a2] + qv * sm
            return carry

        lax.fori_loop(0, OUT // LANES, chunk_body, 0)

    fire(0, rows0_v, sem0)

    def body(h, carry):
        g0 = 2 * h
        g1 = 2 * h + 1

        @pl.when(g1 < nb)
        def _():
            fire(g1, rows1_v, sem1)
        drain(g0, rows0_v, sem0)
        process(g0, rows0_v)

        @pl.when(g1 < nb)
        def _():
            @pl.when(g1 + 1 < nb)
            def _():
                fire(g1 + 1, rows0_v, sem0)
            drain(g1, rows1_v, sem1)
            process(g1, rows1_v)
        return carry

    lax.fori_loop(0, (nb + 1) // 2, body, 0)
    pltpu.sync_copy(m_v, m_hbm.at[pl.ds(base, SEG_PER_W)])
    pltpu.sync_copy(acc_v, part_hbm.at[pl.ds(wid * 3 * OUT, 3 * OUT)])


def _fin_body(m_ref, q_ref, p_ref, out_ref):
    m = m_ref[...]
    q = q_ref[0]
    p = p_ref[...]
    tsum = jnp.sum(p[:, 0, :], axis=0, keepdims=True)
    psq = jnp.sum(p[:, 1, :], axis=0, keepdims=True)
    cross = jnp.sum(p[:, 2, :], axis=0, keepdims=True)
    qs = jnp.sum(q, axis=0, keepdims=True)
    qs2 = jnp.sum(q * q, axis=0, keepdims=True)
    cnt = jnp.float32(N * K)
    mean = (tsum + K * qs) / cnt
    var = (psq + 2.0 * cross + K * qs2) / cnt - mean * mean
    inv = 1.0 / jnp.sqrt(var + 1e-5)
    z = (m + q - mean) * inv
    out_ref[0] = jnp.where(z >= 0, z, 0.2 * z)


_fin_call = pl.pallas_call(
    _fin_body,
    in_specs=[
        pl.BlockSpec((N, OUT), lambda: (0, 0)),
        pl.BlockSpec((1, N, OUT), lambda: (0, 0, 0)),
        pl.BlockSpec((NW, 3, OUT), lambda: (0, 0, 0)),
    ],
    out_specs=pl.BlockSpec((1, N, OUT), lambda: (0, 0, 0)),
    out_shape=jax.ShapeDtypeStruct((1, N, OUT), jnp.float32),
)


def kernel(cloud, W):
    outs = []
    for b in range(B):
        xb = lax.slice_in_dim(cloud, b, b + 1, axis=0)
        pt, qt, idx = _knn_call(xb, xb, W)
        m, parts_flat = _sc_gather(
            pt.reshape(N, OUT), idx.reshape(N * K), qt.reshape(N, OUT))
        outs.append(_fin_call(m, qt, parts_flat.reshape(NW, 3, OUT)))
    out = jnp.concatenate(outs, axis=0)
    return jnp.transpose(out, (0, 2, 1))
